# baseline (speedup 1.0000x reference)
import functools

import jax
import jax.numpy as jnp
from jax import lax
from jax.experimental import pallas as pl
from jax.experimental.pallas import tpu as pltpu
from jax.experimental.pallas import tpu_sc as plsc

DEPTH = 4


def _mm_first(f_bonds, W_i, W_h, blk=1280):
    nb, bd = f_bonds.shape
    h = W_i.shape[1]

    def body(fb_ref, wi_ref, wh_ref, inp_ref, mh_ref):
        inp = jnp.dot(fb_ref[...], wi_ref[...], preferred_element_type=jnp.float32)
        inp_ref[...] = inp
        mh_ref[...] = jnp.dot(jnp.maximum(inp, 0.0), wh_ref[...],
                              preferred_element_type=jnp.float32)

    return pl.pallas_call(
        body,
        grid=(nb // blk,),
        in_specs=[
            pl.BlockSpec((blk, bd), lambda i: (i, 0)),
            pl.BlockSpec((bd, h), lambda i: (0, 0)),
            pl.BlockSpec((h, h), lambda i: (0, 0)),
        ],
        out_specs=[
            pl.BlockSpec((blk, h), lambda i: (i, 0)),
            pl.BlockSpec((blk, h), lambda i: (i, 0)),
        ],
        out_shape=[
            jax.ShapeDtypeStruct((nb, h), jnp.float32),
            jax.ShapeDtypeStruct((nb, h), jnp.float32),
        ],
    )(f_bonds, W_i, W_h)


def _mm_matmul(msg, W_h, blk=1280):
    nb, h = msg.shape

    def body(m_ref, wh_ref, out_ref):
        out_ref[...] = jnp.dot(m_ref[...], wh_ref[...],
                               preferred_element_type=jnp.float32)

    return pl.pallas_call(
        body,
        grid=(nb // blk,),
        in_specs=[
            pl.BlockSpec((blk, h), lambda i: (i, 0)),
            pl.BlockSpec((h, h), lambda i: (0, 0)),
        ],
        out_specs=pl.BlockSpec((blk, h), lambda i: (i, 0)),
        out_shape=jax.ShapeDtypeStruct((nb, h), jnp.float32),
    )(msg, W_h)


def _mm_out(f_atoms, a_msg, W_o1, W_o2, b_o, blk=2000):
    na, fd = f_atoms.shape
    h = W_o2.shape[1]
    b2d = b_o.reshape(1, h)

    def body(fa_ref, am_ref, w1_ref, w2_ref, b_ref, out_ref):
        acc = jnp.dot(fa_ref[...], w1_ref[...], preferred_element_type=jnp.float32)
        acc = acc + jnp.dot(am_ref[...], w2_ref[...], preferred_element_type=jnp.float32)
        out_ref[...] = jnp.maximum(acc + b_ref[...], 0.0)

    return pl.pallas_call(
        body,
        grid=(na // blk,),
        in_specs=[
            pl.BlockSpec((blk, fd), lambda i: (i, 0)),
            pl.BlockSpec((blk, h), lambda i: (i, 0)),
            pl.BlockSpec((fd, h), lambda i: (0, 0)),
            pl.BlockSpec((h, h), lambda i: (0, 0)),
            pl.BlockSpec((1, h), lambda i: (0, 0)),
        ],
        out_specs=pl.BlockSpec((blk, h), lambda i: (i, 0)),
        out_shape=jax.ShapeDtypeStruct((na, h), jnp.float32),
    )(f_atoms, a_msg, W_o1, W_o2, b2d)


def _gather_sum(table, a2bT_flat, n_pad, max_nb, nc, ns, ac=80):
    h = table.shape[1]
    nw = nc * ns
    cs = n_pad // (nw * ac)
    mesh = plsc.VectorSubcoreMesh(core_axis_name="c", subcore_axis_name="s")

    @functools.partial(
        pl.kernel,
        mesh=mesh,
        out_type=jax.ShapeDtypeStruct((n_pad, h), jnp.float32),
        scratch_types=[
            pltpu.VMEM((ac,), jnp.int32),
            pltpu.VMEM((ac, h), jnp.float32),
            pltpu.VMEM((ac, h), jnp.float32),
            pltpu.SemaphoreType.DMA,
        ],
    )
    def k(table_hbm, a2bT_hbm, out_hbm, idx_v, rows_v, acc_v, sem):
        c = lax.axis_index("c")
        s = lax.axis_index("s")
        w = c * ns + s

        def chunk(kk, _):
            atom_base = (w * cs + kk) * ac
            pltpu.sync_copy(a2bT_hbm.at[pl.ds(atom_base, ac)], idx_v)
            pltpu.async_copy(table_hbm.at[idx_v], acc_v, sem).wait()

            def slot(g, _):
                off = g * n_pad + atom_base
                pltpu.sync_copy(a2bT_hbm.at[pl.ds(off, ac)], idx_v)
                pltpu.async_copy(table_hbm.at[idx_v], rows_v, sem).wait()

                def row(r, _):
                    for d in range(h // 16):
                        sl = pl.ds(d * 16, 16)
                        acc_v[r, sl] = acc_v[r, sl] + rows_v[r, sl]
                    return 0

                lax.fori_loop(0, ac, row, 0)
                return 0

            lax.fori_loop(1, max_nb, slot, 0)
            pltpu.sync_copy(acc_v, out_hbm.at[pl.ds(atom_base, ac)])
            return 0

        lax.fori_loop(0, cs, chunk, 0)

    return k(table, a2bT_flat)


def _gather_msg(inp, mh, amh, b2a, b2revb, nc, ns, chunk_rows=80):
    nb, h = inp.shape
    nw = nc * ns
    pw = nb // nw
    nch = pw // chunk_rows
    mesh = plsc.VectorSubcoreMesh(core_axis_name="c", subcore_axis_name="s")

    @functools.partial(
        pl.kernel,
        mesh=mesh,
        out_type=jax.ShapeDtypeStruct((nb, h), jnp.float32),
        scratch_types=[
            pltpu.VMEM((chunk_rows,), jnp.int32),
            pltpu.VMEM((chunk_rows,), jnp.int32),
            pltpu.VMEM((chunk_rows, h), jnp.float32),
            pltpu.VMEM((chunk_rows, h), jnp.float32),
            pltpu.VMEM((chunk_rows, h), jnp.float32),
            pltpu.SemaphoreType.DMA,
            pltpu.SemaphoreType.DMA,
        ],
    )
    def k(inp_hbm, mh_hbm, amh_hbm, b2a_hbm, b2revb_hbm, out_hbm,
          idx1_v, idx2_v, buf_i, buf_a, buf_b, sem, sem2):
        c = lax.axis_index("c")
        s = lax.axis_index("s")
        w = c * ns + s

        def chunk(kk, _):
            base = pl.multiple_of(w * pw + kk * chunk_rows, 8)
            pltpu.sync_copy(b2a_hbm.at[pl.ds(base, chunk_rows)], idx1_v)
            pltpu.sync_copy(b2revb_hbm.at[pl.ds(base, chunk_rows)], idx2_v)
            cp_i = pltpu.async_copy(inp_hbm.at[pl.ds(base, chunk_rows)],
                                    buf_i, sem2)
            cp_a = pltpu.async_copy(amh_hbm.at[idx1_v], buf_a, sem)
            cp_b = pltpu.async_copy(mh_hbm.at[idx2_v], buf_b, sem)
            cp_i.wait()
            cp_a.wait()
            cp_b.wait()

            def row(r, _):
                for d in range(h // 16):
                    sl = pl.ds(d * 16, 16)
                    v = buf_i[r, sl] + buf_a[r, sl] - buf_b[r, sl]
                    buf_i[r, sl] = jnp.maximum(v, 0.0)
                return 0

            lax.fori_loop(0, chunk_rows, row, 0)
            pltpu.sync_copy(buf_i, out_hbm.at[pl.ds(base, chunk_rows)])
            return 0

        lax.fori_loop(0, nch, chunk, 0)

    return k(inp, mh, amh, b2a, b2revb)


def kernel(f_atoms, f_bonds, a2b, b2a, b2revb, W_i, W_h, W_o, b_o):
    n_atoms, max_nb = a2b.shape
    fd = f_atoms.shape[1]

    info = plsc.get_sparse_core_info()
    nc, ns = info.num_cores, info.num_subcores
    nw = nc * ns
    ac = 80

    grp = nw * ac
    n_pad = ((n_atoms + grp - 1) // grp) * grp

    b2a = b2a.astype(jnp.int32)
    b2revb = b2revb.astype(jnp.int32)
    a2bT_flat = jnp.pad(a2b.astype(jnp.int32),
                        ((0, n_pad - n_atoms), (0, 0))).T.reshape(-1)

    inp, mh = _mm_first(f_bonds, W_i, W_h)
    for t in range(DEPTH - 1):
        amh = _gather_sum(mh, a2bT_flat, n_pad, max_nb, nc, ns, ac)
        msg = _gather_msg(inp, mh, amh, b2a, b2revb, nc, ns)
        if t < DEPTH - 2:
            mh = _mm_matmul(msg, W_h)

    a_msg = _gather_sum(msg, a2bT_flat, n_pad, max_nb, nc, ns, ac)[:n_atoms]
    return _mm_out(f_atoms, a_msg, W_o[:fd], W_o[fd:], b_o)


# speedup vs baseline: 1.3878x; 1.3878x over previous
import functools

import jax
import jax.numpy as jnp
from jax import lax
from jax.experimental import pallas as pl
from jax.experimental.pallas import tpu as pltpu
from jax.experimental.pallas import tpu_sc as plsc

DEPTH = 4


def _mm_first(f_bonds, W_i, W_h, blk=1280):
    nb, bd = f_bonds.shape
    h = W_i.shape[1]

    def body(fb_ref, wi_ref, wh_ref, inp_ref, mh_ref):
        inp = jnp.dot(fb_ref[...], wi_ref[...], preferred_element_type=jnp.float32)
        inp_ref[...] = inp
        mh_ref[...] = jnp.dot(jnp.maximum(inp, 0.0), wh_ref[...],
                              preferred_element_type=jnp.float32)

    return pl.pallas_call(
        body,
        grid=(nb // blk,),
        in_specs=[
            pl.BlockSpec((blk, bd), lambda i: (i, 0)),
            pl.BlockSpec((bd, h), lambda i: (0, 0)),
            pl.BlockSpec((h, h), lambda i: (0, 0)),
        ],
        out_specs=[
            pl.BlockSpec((blk, h), lambda i: (i, 0)),
            pl.BlockSpec((blk, h), lambda i: (i, 0)),
        ],
        out_shape=[
            jax.ShapeDtypeStruct((nb, h), jnp.float32),
            jax.ShapeDtypeStruct((nb, h), jnp.float32),
        ],
    )(f_bonds, W_i, W_h)


def _mm_matmul(x, W, blk=1280):
    n, h = x.shape
    ho = W.shape[1]

    def body(x_ref, w_ref, out_ref):
        out_ref[...] = jnp.dot(x_ref[...], w_ref[...],
                               preferred_element_type=jnp.float32)

    return pl.pallas_call(
        body,
        grid=(n // blk,),
        in_specs=[
            pl.BlockSpec((blk, h), lambda i: (i, 0)),
            pl.BlockSpec((h, ho), lambda i: (0, 0)),
        ],
        out_specs=pl.BlockSpec((blk, ho), lambda i: (i, 0)),
        out_shape=jax.ShapeDtypeStruct((n, ho), jnp.float32),
    )(x, W)


def _mm_out1(f_atoms, W_o1, b_o, blk=2000):
    na, fd = f_atoms.shape
    h = W_o1.shape[1]
    b2d = b_o.reshape(1, h)

    def body(fa_ref, w1_ref, b_ref, out_ref):
        out_ref[...] = jnp.dot(fa_ref[...], w1_ref[...],
                               preferred_element_type=jnp.float32) + b_ref[...]

    return pl.pallas_call(
        body,
        grid=(na // blk,),
        in_specs=[
            pl.BlockSpec((blk, fd), lambda i: (i, 0)),
            pl.BlockSpec((fd, h), lambda i: (0, 0)),
            pl.BlockSpec((1, h), lambda i: (0, 0)),
        ],
        out_specs=pl.BlockSpec((blk, h), lambda i: (i, 0)),
        out_shape=jax.ShapeDtypeStruct((na, h), jnp.float32),
    )(f_atoms, W_o1, b2d)


def _mm_out2(P, a_msg, W_o2, blk=2000):
    na, h = P.shape

    def body(p_ref, am_ref, w2_ref, out_ref):
        acc = p_ref[...] + jnp.dot(am_ref[...], w2_ref[...],
                                   preferred_element_type=jnp.float32)
        out_ref[...] = jnp.maximum(acc, 0.0)

    return pl.pallas_call(
        body,
        grid=(na // blk,),
        in_specs=[
            pl.BlockSpec((blk, h), lambda i: (i, 0)),
            pl.BlockSpec((blk, h), lambda i: (i, 0)),
            pl.BlockSpec((h, h), lambda i: (0, 0)),
        ],
        out_specs=pl.BlockSpec((blk, h), lambda i: (i, 0)),
        out_shape=jax.ShapeDtypeStruct((na, h), jnp.float32),
    )(P, a_msg, W_o2)


def _gather_sum(table, a2bT_flat, n_pad, max_nb, nc, ns, ac=80):
    h = table.shape[1]
    nw = nc * ns
    cs = n_pad // (nw * ac)
    mesh = plsc.VectorSubcoreMesh(core_axis_name="c", subcore_axis_name="s")

    @functools.partial(
        pl.kernel,
        mesh=mesh,
        out_type=jax.ShapeDtypeStruct((n_pad, h), jnp.float32),
        scratch_types=[
            pltpu.VMEM((ac,), jnp.int32),
            pltpu.VMEM((ac,), jnp.int32),
            pltpu.VMEM((ac,), jnp.int32),
            pltpu.VMEM((ac, h), jnp.float32),
            pltpu.VMEM((ac, h), jnp.float32),
            pltpu.VMEM((ac, h), jnp.float32),
            pltpu.SemaphoreType.DMA,
            pltpu.SemaphoreType.DMA,
            pltpu.SemaphoreType.DMA,
            pltpu.SemaphoreType.DMA,
        ],
    )
    def k(table_hbm, a2bT_hbm, out_hbm, idx_a, idx_0, idx_1, acc_v, rows_0,
          rows_1, sem_a, sem_0, sem_1, sem_o):
        c = lax.axis_index("c")
        s = lax.axis_index("s")
        w = c * ns + s
        idx = (idx_0, idx_1)
        rows = (rows_0, rows_1)
        sems = (sem_0, sem_1)

        def chunk(kk, _):
            atom_base = (w * cs + kk) * ac

            @pl.when(kk > 0)
            def _():
                pltpu.make_async_copy(acc_v, out_hbm.at[pl.ds(0, ac)],
                                      sem_o).wait()

            pltpu.sync_copy(a2bT_hbm.at[pl.ds(atom_base, ac)], idx_a)
            pltpu.async_copy(table_hbm.at[idx_a], acc_v, sem_a)
            pltpu.sync_copy(a2bT_hbm.at[pl.ds(n_pad + atom_base, ac)], idx[1])
            pltpu.async_copy(table_hbm.at[idx[1]], rows[1], sems[1])
            pltpu.make_async_copy(table_hbm.at[idx_a], acc_v, sem_a).wait()

            for g in range(1, max_nb):
                b = g % 2
                if g + 1 < max_nb:
                    nb_ = (g + 1) % 2
                    off = (g + 1) * n_pad + atom_base
                    pltpu.sync_copy(a2bT_hbm.at[pl.ds(off, ac)], idx[nb_])
                    pltpu.async_copy(table_hbm.at[idx[nb_]], rows[nb_],
                                     sems[nb_])
                pltpu.make_async_copy(table_hbm.at[idx[b]], rows[b],
                                      sems[b]).wait()
                rbuf = rows[b]

                def row(r, _):
                    for d in range(h // 16):
                        sl = pl.ds(d * 16, 16)
                        acc_v[r, sl] = acc_v[r, sl] + rbuf[r, sl]
                    return 0

                lax.fori_loop(0, ac, row, 0)

            pltpu.async_copy(acc_v, out_hbm.at[pl.ds(atom_base, ac)], sem_o)
            return 0

        lax.fori_loop(0, cs, chunk, 0)
        pltpu.make_async_copy(acc_v, out_hbm.at[pl.ds(0, ac)], sem_o).wait()

    return k(table, a2bT_flat)


def _gather_msg(inp, mh, amh, b2a, b2revb, nc, ns, cr=40):
    nb, h = inp.shape
    nw = nc * ns
    pw = nb // nw
    npair = pw // (2 * cr)
    mesh = plsc.VectorSubcoreMesh(core_axis_name="c", subcore_axis_name="s")

    @functools.partial(
        pl.kernel,
        mesh=mesh,
        out_type=jax.ShapeDtypeStruct((nb, h), jnp.float32),
        scratch_types=[
            pltpu.VMEM((cr,), jnp.int32),
            pltpu.VMEM((cr,), jnp.int32),
            pltpu.VMEM((cr,), jnp.int32),
            pltpu.VMEM((cr,), jnp.int32),
            pltpu.VMEM((cr, h), jnp.float32),
            pltpu.VMEM((cr, h), jnp.float32),
            pltpu.VMEM((cr, h), jnp.float32),
            pltpu.VMEM((cr, h), jnp.float32),
            pltpu.VMEM((cr, h), jnp.float32),
            pltpu.VMEM((cr, h), jnp.float32),
            pltpu.SemaphoreType.DMA,
            pltpu.SemaphoreType.DMA,
            pltpu.SemaphoreType.DMA,
            pltpu.SemaphoreType.DMA,
        ],
    )
    def k(inp_hbm, mh_hbm, amh_hbm, b2a_hbm, b2revb_hbm, out_hbm,
          i1_0, i1_1, i2_0, i2_1, bi_0, bi_1, ba_0, ba_1, bb_0, bb_1,
          semi_0, semi_1, semo_0, semo_1):
        c = lax.axis_index("c")
        s = lax.axis_index("s")
        w = c * ns + s
        i1 = (i1_0, i1_1)
        i2 = (i2_0, i2_1)
        bi = (bi_0, bi_1)
        ba = (ba_0, ba_1)
        bb = (bb_0, bb_1)
        semi = (semi_0, semi_1)
        semo = (semo_0, semo_1)

        def issue_in(cc, sl):
            base = pl.multiple_of(w * pw + cc * cr, 8)
            pltpu.sync_copy(b2a_hbm.at[pl.ds(base, cr)], i1[sl])
            pltpu.sync_copy(b2revb_hbm.at[pl.ds(base, cr)], i2[sl])
            pltpu.async_copy(inp_hbm.at[pl.ds(base, cr)], bi[sl], semi[sl])
            pltpu.async_copy(amh_hbm.at[i1[sl]], ba[sl], semi[sl])
            pltpu.async_copy(mh_hbm.at[i2[sl]], bb[sl], semi[sl])

        def wait_in(sl):
            for _ in range(3):
                pltpu.make_async_copy(inp_hbm.at[pl.ds(0, cr)], bi[sl],
                                      semi[sl]).wait()

        def vpu(sl):
            bis, bas, bbs = bi[sl], ba[sl], bb[sl]

            def row(r, _):
                for d in range(h // 16):
                    sl_ = pl.ds(d * 16, 16)
                    v = bis[r, sl_] + bas[r, sl_] - bbs[r, sl_]
                    bis[r, sl_] = jnp.maximum(v, 0.0)
                return 0

            lax.fori_loop(0, cr, row, 0)

        def issue_out(cc, sl):
            base = pl.multiple_of(w * pw + cc * cr, 8)
            pltpu.async_copy(bi[sl], out_hbm.at[pl.ds(base, cr)], semo[sl])

        def wait_out(sl):
            pltpu.make_async_copy(bi[sl], out_hbm.at[pl.ds(0, cr)],
                                  semo[sl]).wait()

        issue_in(0, 0)

        def body(kk, _):
            c0 = 2 * kk
            c1 = 2 * kk + 1

            @pl.when(kk > 0)
            def _():
                wait_out(1)

            issue_in(c1, 1)
            wait_in(0)
            vpu(0)
            issue_out(c0, 0)
            wait_in(1)

            @pl.when(kk + 1 < npair)
            def _():
                wait_out(0)
                issue_in(c0 + 2, 0)

            vpu(1)
            issue_out(c1, 1)
            return 0

        lax.fori_loop(0, npair, body, 0)
        wait_out(0)
        wait_out(1)

    return k(inp, mh, amh, b2a, b2revb)


def kernel(f_atoms, f_bonds, a2b, b2a, b2revb, W_i, W_h, W_o, b_o):
    n_atoms, max_nb = a2b.shape
    fd = f_atoms.shape[1]

    info = plsc.get_sparse_core_info()
    nc, ns = info.num_cores, info.num_subcores
    nw = nc * ns
    ac = 80

    grp = nw * ac
    n_pad = ((n_atoms + grp - 1) // grp) * grp

    b2a = b2a.astype(jnp.int32)
    b2revb = b2revb.astype(jnp.int32)
    a2bT_flat = jnp.pad(a2b.astype(jnp.int32),
                        ((0, n_pad - n_atoms), (0, 0))).T.reshape(-1)

    P = _mm_out1(f_atoms, W_o[:fd], b_o)
    inp, mh = _mm_first(f_bonds, W_i, W_h)
    amh = _gather_sum(mh, a2bT_flat, n_pad, max_nb, nc, ns, ac)
    msg = _gather_msg(inp, mh, amh, b2a, b2revb, nc, ns)
    for _ in range(DEPTH - 2):
        mh = _mm_matmul(msg, W_h)
        asum = _gather_sum(msg, a2bT_flat, n_pad, max_nb, nc, ns, ac)
        amh = _mm_matmul(asum, W_h)
        msg = _gather_msg(inp, mh, amh, b2a, b2revb, nc, ns)

    a_msg = _gather_sum(msg, a2bT_flat, n_pad, max_nb, nc, ns, ac)[:n_atoms]
    return _mm_out2(P, a_msg, W_o[fd:])
